# baseline (speedup 1.0000x reference)
import functools

import jax
import jax.numpy as jnp
from jax import lax
from jax.experimental import pallas as pl
from jax.experimental.pallas import tpu as pltpu
from jax.experimental.pallas import tpu_sc as plsc

BATCH = 16384
D = 64
V = 1000000
NC = 2
NS = 16
NW = NC * NS
B_PER_W = BATCH // NW
CHUNK = 128
NCHUNK = B_PER_W // CHUNK
RB = 6144


def _transpose_body(t_ref, out_ref):
    r = lax.broadcasted_iota(jnp.int32, (D, D), 0)
    c = lax.broadcasted_iota(jnp.int32, (D, D), 1)
    eye = (r == c).astype(jnp.float32)
    tr = lax.dot_general(
        t_ref[...], eye, (((0,), (0,)), ((), ())),
        preferred_element_type=jnp.float32)
    out_ref[...] = jnp.concatenate((tr, tr), axis=1)


def _tc_pad_transpose(tab_t):
    return pl.pallas_call(
        _transpose_body,
        grid=(pl.cdiv(V, RB),),
        in_specs=[pl.BlockSpec((D, RB), lambda i: (0, i))],
        out_specs=pl.BlockSpec((RB, 128), lambda i: (i, 0)),
        out_shape=jax.ShapeDtypeStruct((V, 128), jnp.float32),
    )(tab_t)


NSTRIP = 7812
SPW = NSTRIP // NW
VMAIN = NSTRIP * 128


def _strip_transpose(in_v, out_v):
    for m in range(64):
        for g in range(8):
            c0 = g * 16
            f0 = c0 % 64
            r = 2 * m + (1 if c0 >= 64 else 0)
            fi = lax.broadcasted_iota(jnp.int32, (16,), 0) + f0
            ri = jnp.full((16,), r, jnp.int32)
            out_v[m, pl.ds(c0, 16)] = plsc.load_gather(in_v, [fi, ri])


def _itr_body(it_hbm, out_hbm, in_a, in_b, out_a, out_b, sem, semo):
    cid = lax.axis_index("c")
    sid = lax.axis_index("s")
    wid = sid * NC + cid

    def strips(k, _):
        s1 = pl.multiple_of((wid + (2 * k) * NW) * 128, 128)
        s2 = pl.multiple_of((wid + (2 * k + 1) * NW) * 128, 128)
        cp_a = pltpu.async_copy(it_hbm.at[:, pl.ds(s1, 128)], in_a, sem)
        cp_b = pltpu.async_copy(it_hbm.at[:, pl.ds(s2, 128)], in_b, sem)
        cp_a.wait()
        _strip_transpose(in_a, out_a)
        oa = pltpu.async_copy(
            out_a, out_hbm.at[pl.ds(pl.multiple_of(s1 // 2, 64), 64), :], semo)
        cp_b.wait()
        _strip_transpose(in_b, out_b)
        ob = pltpu.async_copy(
            out_b, out_hbm.at[pl.ds(pl.multiple_of(s2 // 2, 64), 64), :], semo)
        oa.wait()
        ob.wait()
        return 0

    lax.fori_loop(0, SPW // 2, strips, 0)
    rem = NSTRIP - SPW * NW

    @pl.when(wid < rem)
    def _():
        s = pl.multiple_of((SPW * NW + wid) * 128, 128)
        pltpu.sync_copy(it_hbm.at[:, pl.ds(s, 128)], in_a)
        _strip_transpose(in_a, out_a)
        pltpu.sync_copy(out_a, out_hbm.at[pl.ds(pl.multiple_of(s // 2, 64), 64), :])


def _sc_item_pairs(it_t):
    mesh = plsc.VectorSubcoreMesh(core_axis_name="c", subcore_axis_name="s")
    fn = functools.partial(
        pl.kernel,
        mesh=mesh,
        out_type=jax.ShapeDtypeStruct((VMAIN // 2, 128), jnp.float32),
        scratch_types=[
            pltpu.VMEM((D, 128), jnp.float32),
            pltpu.VMEM((D, 128), jnp.float32),
            pltpu.VMEM((D, 128), jnp.float32),
            pltpu.VMEM((D, 128), jnp.float32),
            pltpu.SemaphoreType.DMA,
            pltpu.SemaphoreType.DMA,
        ],
        compiler_params=pltpu.CompilerParams(needs_layout_passes=False),
    )(_itr_body)
    return fn(it_t)


def _gather_body(u_hbm, gp_hbm, gn_hbm, upad_hbm, ipairs_hbm,
                 wu_out, wp_out, wn_out,
                 gidx, rows, sem):
    cid = lax.axis_index("c")
    sid = lax.axis_index("s")
    wid = sid * NC + cid
    base = pl.multiple_of(wid * B_PER_W, B_PER_W)

    for idx_hbm, table_hbm, out in (
        (u_hbm, upad_hbm, wu_out),
        (gp_hbm, ipairs_hbm, wp_out),
        (gn_hbm, ipairs_hbm, wn_out),
    ):
        pltpu.sync_copy(idx_hbm.at[wid], gidx)
        copies = []
        for c in range(NCHUNK):
            copies.append(pltpu.async_copy(
                table_hbm.at[gidx.at[c]],
                rows.at[pl.ds(c * CHUNK, CHUNK), :], sem))
        for cp in copies:
            cp.wait()
        pltpu.sync_copy(rows, out.at[pl.ds(base, B_PER_W), :])


@jax.jit
def _bprmf_call(user, pos_item, neg_item, user_table, item_table):
    upad = _tc_pad_transpose(user_table.T)
    ipairs = _sc_item_pairs(item_table.T)

    mesh = plsc.VectorSubcoreMesh(core_axis_name="c", subcore_axis_name="s")
    out_w = jax.ShapeDtypeStruct((BATCH, 2 * D), jnp.float32)
    fn = functools.partial(
        pl.kernel,
        mesh=mesh,
        out_type=(out_w, out_w, out_w),
        scratch_types=[
            pltpu.VMEM((NCHUNK, CHUNK), jnp.int32),
            pltpu.VMEM((B_PER_W, 2 * D), jnp.float32),
            pltpu.SemaphoreType.DMA,
        ],
    )(_gather_body)
    u_r = user.reshape(NW, NCHUNK, CHUNK)
    pos_c = jnp.minimum(pos_item, VMAIN - 1)
    neg_c = jnp.minimum(neg_item, VMAIN - 1)
    gp = (pos_c >> 1).reshape(NW, NCHUNK, CHUNK)
    gn = (neg_c >> 1).reshape(NW, NCHUNK, CHUNK)
    wu, wp, wn = fn(u_r, gp, gn, upad, ipairs)

    tail = item_table[VMAIN:]

    def half_select(wide, idx, idx_c):
        odd = (idx_c & 1).astype(jnp.bool_)[:, None]
        main = jnp.where(odd, wide[:, D:], wide[:, :D])
        fix = tail[jnp.clip(idx - VMAIN, 0, V - VMAIN - 1)]
        return jnp.where((idx >= VMAIN)[:, None], fix, main)

    return (wu[:, :D],
            half_select(wp, pos_item, pos_c),
            half_select(wn, neg_item, neg_c))


def kernel(user, pos_item, neg_item, user_table, item_table):
    return _bprmf_call(user, pos_item, neg_item, user_table, item_table)


# speedup vs baseline: 2.1456x; 2.1456x over previous
import functools

import jax
import jax.numpy as jnp
from jax import lax
from jax.experimental import pallas as pl
from jax.experimental.pallas import tpu as pltpu
from jax.experimental.pallas import tpu_sc as plsc

BATCH = 16384
D = 64
V = 1000000
NC = 2
NS = 16
NW = NC * NS
B_PER_W = BATCH // NW
CHUNK = 128
NCHUNK = B_PER_W // CHUNK
RB = 6144


def _transpose_body(t_ref, out_ref):
    r = lax.broadcasted_iota(jnp.int32, (D, 2 * D), 0)
    c = lax.broadcasted_iota(jnp.int32, (D, 2 * D), 1)
    eye2 = (r == c % D).astype(jnp.float32)
    out_ref[...] = lax.dot_general(
        t_ref[...], eye2, (((0,), (0,)), ((), ())),
        preferred_element_type=jnp.float32)


def _tc_pad_transpose(tab_t):
    return pl.pallas_call(
        _transpose_body,
        grid=(pl.cdiv(V, RB),),
        in_specs=[pl.BlockSpec((D, RB), lambda i: (0, i))],
        out_specs=pl.BlockSpec((RB, 128), lambda i: (i, 0)),
        out_shape=jax.ShapeDtypeStruct((V, 128), jnp.float32),
    )(tab_t)


def _gather_body(u_hbm, gp_hbm, gn_hbm, upad_hbm, ipairs_hbm,
                 wu_out, wp_out, wn_out,
                 gidx, rows, sem):
    cid = lax.axis_index("c")
    sid = lax.axis_index("s")
    wid = sid * NC + cid
    base = pl.multiple_of(wid * B_PER_W, B_PER_W)

    for idx_hbm, table_hbm, out in (
        (u_hbm, upad_hbm, wu_out),
        (gp_hbm, ipairs_hbm, wp_out),
        (gn_hbm, ipairs_hbm, wn_out),
    ):
        pltpu.sync_copy(idx_hbm.at[wid], gidx)
        copies = []
        for c in range(NCHUNK):
            copies.append(pltpu.async_copy(
                table_hbm.at[gidx.at[c]],
                rows.at[pl.ds(c * CHUNK, CHUNK), :], sem))
        for cp in copies:
            cp.wait()
        pltpu.sync_copy(rows, out.at[pl.ds(base, B_PER_W), :])


@jax.jit
def _bprmf_call(user, pos_item, neg_item, user_table, item_table):
    upad = _tc_pad_transpose(user_table.T)
    ipairs = item_table.reshape(V // 2, 2 * D)

    mesh = plsc.VectorSubcoreMesh(core_axis_name="c", subcore_axis_name="s")
    out_w = jax.ShapeDtypeStruct((BATCH, 2 * D), jnp.float32)
    fn = functools.partial(
        pl.kernel,
        mesh=mesh,
        out_type=(out_w, out_w, out_w),
        scratch_types=[
            pltpu.VMEM((NCHUNK, CHUNK), jnp.int32),
            pltpu.VMEM((B_PER_W, 2 * D), jnp.float32),
            pltpu.SemaphoreType.DMA,
        ],
    )(_gather_body)
    u_r = user.reshape(NW, NCHUNK, CHUNK)
    gp = (pos_item >> 1).reshape(NW, NCHUNK, CHUNK)
    gn = (neg_item >> 1).reshape(NW, NCHUNK, CHUNK)
    wu, wp, wn = fn(u_r, gp, gn, upad, ipairs)

    def half_select(wide, idx):
        odd = (idx & 1).astype(jnp.bool_)[:, None]
        return jnp.where(odd, wide[:, D:], wide[:, :D])

    return (wu[:, :D],
            half_select(wp, pos_item),
            half_select(wn, neg_item))


def kernel(user, pos_item, neg_item, user_table, item_table):
    return _bprmf_call(user, pos_item, neg_item, user_table, item_table)


# speedup vs baseline: 2.1987x; 1.0248x over previous
import functools

import jax
import jax.numpy as jnp
from jax import lax
from jax.experimental import pallas as pl
from jax.experimental.pallas import tpu as pltpu
from jax.experimental.pallas import tpu_sc as plsc

BATCH = 16384
D = 64
V = 1000000
NC = 2
NS = 16
NW = NC * NS
B_PER_W = BATCH // NW
CHUNK = 128
NCHUNK = B_PER_W // CHUNK
RB = 12288


def _transpose_body(t_ref, out_ref):
    r = lax.broadcasted_iota(jnp.int32, (D, 2 * D), 0)
    c = lax.broadcasted_iota(jnp.int32, (D, 2 * D), 1)
    eye2 = (r == c % D).astype(jnp.float32)
    out_ref[...] = lax.dot_general(
        t_ref[...], eye2, (((0,), (0,)), ((), ())),
        preferred_element_type=jnp.float32)


def _tc_pad_transpose(tab_t):
    return pl.pallas_call(
        _transpose_body,
        grid=(pl.cdiv(V, RB),),
        in_specs=[pl.BlockSpec((D, RB), lambda i: (0, i))],
        out_specs=pl.BlockSpec((RB, 128), lambda i: (i, 0)),
        out_shape=jax.ShapeDtypeStruct((V, 128), jnp.float32),
    )(tab_t)


def _gather_body(u_hbm, gp_hbm, gn_hbm, upad_hbm, ipairs_hbm,
                 wu_out, wp_out, wn_out,
                 gidx, rows, sem):
    cid = lax.axis_index("c")
    sid = lax.axis_index("s")
    wid = sid * NC + cid
    base = pl.multiple_of(wid * B_PER_W, B_PER_W)

    for idx_hbm, table_hbm, out in (
        (u_hbm, upad_hbm, wu_out),
        (gp_hbm, ipairs_hbm, wp_out),
        (gn_hbm, ipairs_hbm, wn_out),
    ):
        pltpu.sync_copy(idx_hbm.at[wid], gidx)
        copies = []
        for c in range(NCHUNK):
            copies.append(pltpu.async_copy(
                table_hbm.at[gidx.at[c]],
                rows.at[pl.ds(c * CHUNK, CHUNK), :], sem))
        for cp in copies:
            cp.wait()
        pltpu.sync_copy(rows, out.at[pl.ds(base, B_PER_W), :])


@jax.jit
def _bprmf_call(user, pos_item, neg_item, user_table, item_table):
    upad = _tc_pad_transpose(user_table.T)
    ipairs = item_table.reshape(V // 2, 2 * D)

    mesh = plsc.VectorSubcoreMesh(core_axis_name="c", subcore_axis_name="s")
    out_w = jax.ShapeDtypeStruct((BATCH, 2 * D), jnp.float32)
    fn = functools.partial(
        pl.kernel,
        mesh=mesh,
        out_type=(out_w, out_w, out_w),
        scratch_types=[
            pltpu.VMEM((NCHUNK, CHUNK), jnp.int32),
            pltpu.VMEM((B_PER_W, 2 * D), jnp.float32),
            pltpu.SemaphoreType.DMA,
        ],
    )(_gather_body)
    u_r = user.reshape(NW, NCHUNK, CHUNK)
    gp = (pos_item >> 1).reshape(NW, NCHUNK, CHUNK)
    gn = (neg_item >> 1).reshape(NW, NCHUNK, CHUNK)
    wu, wp, wn = fn(u_r, gp, gn, upad, ipairs)

    def half_select(wide, idx):
        odd = (idx & 1).astype(jnp.bool_)[:, None]
        return jnp.where(odd, wide[:, D:], wide[:, :D])

    return (wu[:, :D],
            half_select(wp, pos_item),
            half_select(wn, neg_item))


def kernel(user, pos_item, neg_item, user_table, item_table):
    return _bprmf_call(user, pos_item, neg_item, user_table, item_table)


# speedup vs baseline: 2.2146x; 1.0072x over previous
import functools

import jax
import jax.numpy as jnp
from jax import lax
from jax.experimental import pallas as pl
from jax.experimental.pallas import tpu as pltpu
from jax.experimental.pallas import tpu_sc as plsc

BATCH = 16384
D = 64
V = 1000000
NC = 2
NS = 16
NW = NC * NS
B_PER_W = BATCH // NW
CHUNK = 128
NCHUNK = B_PER_W // CHUNK
RB = 24576


def _transpose_body(t_ref, out_ref):
    r = lax.broadcasted_iota(jnp.int32, (D, 2 * D), 0)
    c = lax.broadcasted_iota(jnp.int32, (D, 2 * D), 1)
    eye2 = (r == c % D).astype(jnp.float32)
    out_ref[...] = lax.dot_general(
        t_ref[...], eye2, (((0,), (0,)), ((), ())),
        preferred_element_type=jnp.float32)


def _tc_pad_transpose(tab_t):
    return pl.pallas_call(
        _transpose_body,
        grid=(pl.cdiv(V, RB),),
        in_specs=[pl.BlockSpec((D, RB), lambda i: (0, i))],
        out_specs=pl.BlockSpec((RB, 128), lambda i: (i, 0)),
        out_shape=jax.ShapeDtypeStruct((V, 128), jnp.float32),
    )(tab_t)


def _gather_body(u_hbm, gp_hbm, gn_hbm, upad_hbm, ipairs_hbm,
                 wu_out, wp_out, wn_out,
                 gidx, rows, sem):
    cid = lax.axis_index("c")
    sid = lax.axis_index("s")
    wid = sid * NC + cid
    base = pl.multiple_of(wid * B_PER_W, B_PER_W)

    for idx_hbm, table_hbm, out in (
        (u_hbm, upad_hbm, wu_out),
        (gp_hbm, ipairs_hbm, wp_out),
        (gn_hbm, ipairs_hbm, wn_out),
    ):
        pltpu.sync_copy(idx_hbm.at[wid], gidx)
        copies = []
        for c in range(NCHUNK):
            copies.append(pltpu.async_copy(
                table_hbm.at[gidx.at[c]],
                rows.at[pl.ds(c * CHUNK, CHUNK), :], sem))
        for cp in copies:
            cp.wait()
        pltpu.sync_copy(rows, out.at[pl.ds(base, B_PER_W), :])


@jax.jit
def _bprmf_call(user, pos_item, neg_item, user_table, item_table):
    upad = _tc_pad_transpose(user_table.T)
    ipairs = item_table.reshape(V // 2, 2 * D)

    mesh = plsc.VectorSubcoreMesh(core_axis_name="c", subcore_axis_name="s")
    out_w = jax.ShapeDtypeStruct((BATCH, 2 * D), jnp.float32)
    fn = functools.partial(
        pl.kernel,
        mesh=mesh,
        out_type=(out_w, out_w, out_w),
        scratch_types=[
            pltpu.VMEM((NCHUNK, CHUNK), jnp.int32),
            pltpu.VMEM((B_PER_W, 2 * D), jnp.float32),
            pltpu.SemaphoreType.DMA,
        ],
    )(_gather_body)
    u_r = user.reshape(NW, NCHUNK, CHUNK)
    gp = (pos_item >> 1).reshape(NW, NCHUNK, CHUNK)
    gn = (neg_item >> 1).reshape(NW, NCHUNK, CHUNK)
    wu, wp, wn = fn(u_r, gp, gn, upad, ipairs)

    def half_select(wide, idx):
        odd = (idx & 1).astype(jnp.bool_)[:, None]
        return jnp.where(odd, wide[:, D:], wide[:, :D])

    return (wu[:, :D],
            half_select(wp, pos_item),
            half_select(wn, neg_item))


def kernel(user, pos_item, neg_item, user_table, item_table):
    return _bprmf_call(user, pos_item, neg_item, user_table, item_table)


# speedup vs baseline: 2.4625x; 1.1120x over previous
import functools

import jax
import jax.numpy as jnp
from jax import lax
from jax.experimental import pallas as pl
from jax.experimental.pallas import tpu as pltpu
from jax.experimental.pallas import tpu_sc as plsc

BATCH = 16384
D = 64
V = 1000000
NC = 2
NS = 16
NW = NC * NS
B_PER_W = BATCH // NW
CHUNK = 128
NCHUNK = B_PER_W // CHUNK
RB = 24576


def _transpose_body(t_ref, out_ref):
    r = lax.broadcasted_iota(jnp.int32, (D, 2 * D), 0)
    c = lax.broadcasted_iota(jnp.int32, (D, 2 * D), 1)
    eye2 = (r == c % D).astype(jnp.float32)
    out_ref[...] = lax.dot_general(
        t_ref[...], eye2, (((0,), (0,)), ((), ())),
        preferred_element_type=jnp.float32)


def _tc_pad_transpose(tab_t):
    return pl.pallas_call(
        _transpose_body,
        grid=(pl.cdiv(V, RB),),
        in_specs=[pl.BlockSpec((D, RB), lambda i: (0, i))],
        out_specs=pl.BlockSpec((RB, 128), lambda i: (i, 0)),
        out_shape=jax.ShapeDtypeStruct((V, 128), jnp.float32),
    )(tab_t)


def _gather_body(u_hbm, gp_hbm, gn_hbm, upad_hbm, ipairs_hbm,
                 wu_out, wp_out, wn_out,
                 gidx, rows, sem):
    cid = lax.axis_index("c")
    sid = lax.axis_index("s")
    wid = sid * NC + cid
    base = pl.multiple_of(wid * B_PER_W, B_PER_W)

    for idx_hbm, table_hbm, out in (
        (u_hbm, upad_hbm, wu_out),
        (gp_hbm, ipairs_hbm, wp_out),
        (gn_hbm, ipairs_hbm, wn_out),
    ):
        pltpu.sync_copy(idx_hbm.at[wid], gidx)
        copies = []
        for c in range(NCHUNK):
            copies.append(pltpu.async_copy(
                table_hbm.at[gidx.at[c]],
                rows.at[pl.ds(c * CHUNK, CHUNK), :], sem))
        for cp in copies:
            cp.wait()
        pltpu.sync_copy(rows, out.at[pl.ds(base, B_PER_W), :])


@jax.jit
def _bprmf_call(user, pos_item, neg_item, user_table, item_table):
    upad = _tc_pad_transpose(user_table.T)
    ipad = jnp.pad(item_table, ((0, 0), (0, D)))

    mesh = plsc.VectorSubcoreMesh(core_axis_name="c", subcore_axis_name="s")
    out_w = jax.ShapeDtypeStruct((BATCH, 2 * D), jnp.float32)
    fn = functools.partial(
        pl.kernel,
        mesh=mesh,
        out_type=(out_w, out_w, out_w),
        scratch_types=[
            pltpu.VMEM((NCHUNK, CHUNK), jnp.int32),
            pltpu.VMEM((B_PER_W, 2 * D), jnp.float32),
            pltpu.SemaphoreType.DMA,
        ],
    )(_gather_body)
    u_r = user.reshape(NW, NCHUNK, CHUNK)
    gp = pos_item.reshape(NW, NCHUNK, CHUNK)
    gn = neg_item.reshape(NW, NCHUNK, CHUNK)
    wu, wp, wn = fn(u_r, gp, gn, upad, ipad)
    return wu[:, :D], wp[:, :D], wn[:, :D]


def kernel(user, pos_item, neg_item, user_table, item_table):
    return _bprmf_call(user, pos_item, neg_item, user_table, item_table)
